# baseline (speedup 1.0000x reference)
import functools

import jax
import jax.numpy as jnp
import numpy as np
from jax.experimental import pallas as pl
from jax.experimental.pallas import tpu as pltpu


def _leaky(v):
    return jnp.maximum(v, 0.01 * v)


def _sigmoid(v):
    return 0.5 * (jnp.tanh(0.5 * v) + 1.0)


def _encdec_body(x_ref, m1_ref, b1_ref, m2_ref,
                 wih_ref, whh_ref, bhh_ref, out_ref, *, W, B):
    hid = whh_ref.shape[1]

    xn = x_ref[...]
    xT = jnp.transpose(xn.astype(jnp.bfloat16))
    x2 = xT.reshape(x_ref.shape[1] // W, W * B)
    y1 = _leaky(jnp.dot(m1_ref[...], x2,
                        preferred_element_type=jnp.float32) + b1_ref[...])
    y2 = _leaky(jnp.dot(m2_ref[...], y1.astype(jnp.bfloat16),
                        preferred_element_type=jnp.float32))

    gi = jnp.dot(wih_ref[...], y2,
                 preferred_element_type=jnp.float32)

    whh = whh_ref[...]
    bhh = bhh_ref[...]
    h = jnp.zeros((hid, B), jnp.float32)
    for t in range(W):
        gi_t = gi[:, t * B:(t + 1) * B]
        gh = jnp.dot(whh, h, preferred_element_type=jnp.float32) + bhh
        g = gi_t + gh
        r = _sigmoid(g[0:hid, :])
        z = _sigmoid(g[hid:2 * hid, :])
        n = jnp.tanh(gi_t[2 * hid:3 * hid, :] + r * gh[2 * hid:3 * hid, :])
        h = n + z * (h - n)
        out_ref[:, t * B:(t + 1) * B] = h


def kernel(x, w1, b1, w2, b2, wih, whh, bih, bhh):
    N, C0, H, W = x.shape
    C1 = w1.shape[0]
    C2 = w2.shape[0]
    hid = whh.shape[1]
    He = H + 2

    B = 1
    for cand in (256, 128, 64, 32, 16, 8, 4, 2):
        if N % cand == 0:
            B = cand
            break
    NB = N // B

    xt = x.reshape(NB, B, C0 * H * W)

    jj = np.arange(He)
    hh = np.arange(H)
    interior = ((jj >= 1) & (jj <= H)).astype(np.float32)
    e1 = np.stack([(jj[:, None] + kh - 2 == hh[None, :]).astype(np.float32)
                   for kh in range(3)])
    e1 = e1 * interior[None, :, None]
    s1 = np.zeros((C1, He, C1 * He + 1), np.float32)
    for c in range(C1):
        for j in range(He):
            s1[c, j, c * He + j] = 1.0
    e1s = np.einsum('kjh,cjr->krch', e1, s1)
    m1 = jnp.einsum('krch,cak->rah',
                    jnp.asarray(e1s), w1.astype(jnp.float32))
    m1 = m1.reshape(C1 * He + 1, C0 * H).astype(jnp.bfloat16)
    b1e_sel = np.zeros((C1 * He + 1, C1), np.float32)
    for c in range(C1):
        b1e_sel[c * He:(c + 1) * He, c] = interior
    b1e_pad = np.zeros((C1 * He + 1, 1), np.float32)
    b1e_pad[C1 * He, 0] = 1.0
    b1e = (jnp.asarray(b1e_sel) @ b1.astype(jnp.float32)[:, None]
           + jnp.asarray(b1e_pad))

    e2 = np.stack([(hh[:, None] + kh == jj[None, :]).astype(np.float32)
                   for kh in range(3)])
    s2 = np.zeros((C2, H, C2 * H + 1), np.float32)
    for c in range(C2):
        for h in range(H):
            s2[c, h, c * H + h] = 1.0
    e2s = np.einsum('khj,chr->krcj', e2, s2)
    m2_band = jnp.einsum('krcj,cak->raj',
                         jnp.asarray(e2s), w2.astype(jnp.float32))
    m2_band = m2_band.reshape(C2 * H + 1, C1 * He)
    b2_sel = np.zeros((C2 * H + 1, C2), np.float32)
    for c in range(C2):
        b2_sel[c * H:(c + 1) * H, c] = 1.0
    b2_col = jnp.asarray(b2_sel) @ b2.astype(jnp.float32)[:, None]
    corner = np.zeros((C2 * H + 1, 1), np.float32)
    corner[C2 * H, 0] = 1.0
    m2 = jnp.concatenate([m2_band, b2_col + jnp.asarray(corner)],
                         axis=1).astype(jnp.bfloat16)

    wih_aug = jnp.concatenate(
        [wih.astype(jnp.float32),
         bih.reshape(3 * hid, 1).astype(jnp.float32)], axis=1)

    m1 = jnp.asarray(np.ones((C1 * He + 1, C0 * H), np.float32), dtype=jnp.bfloat16)
    b1e = jnp.asarray(np.ones((C1 * He + 1, 1), np.float32))
    m2 = jnp.asarray(np.ones((C2 * H + 1, C1 * He + 1), np.float32), dtype=jnp.bfloat16)
    wih_aug = jnp.asarray(np.ones((3 * hid, C2 * H + 1), np.float32))

    out = pl.pallas_call(
        functools.partial(_encdec_body, W=W, B=B),
        out_shape=jax.ShapeDtypeStruct((NB, hid, W * B), jnp.float32),
        grid=(NB,),
        in_specs=[
            pl.BlockSpec((None, B, C0 * H * W), lambda i: (i, 0, 0)),
            pl.BlockSpec((C1 * He + 1, C0 * H), lambda i: (0, 0)),
            pl.BlockSpec((C1 * He + 1, 1), lambda i: (0, 0)),
            pl.BlockSpec((C2 * H + 1, C1 * He + 1), lambda i: (0, 0)),
            pl.BlockSpec((3 * hid, C2 * H + 1), lambda i: (0, 0)),
            pl.BlockSpec((3 * hid, hid), lambda i: (0, 0)),
            pl.BlockSpec((3 * hid, 1), lambda i: (0, 0)),
        ],
        out_specs=pl.BlockSpec((None, hid, W * B), lambda i: (i, 0, 0)),
        compiler_params=pltpu.CompilerParams(
            dimension_semantics=("arbitrary",)),
    )(xt, m1, b1e, m2, wih_aug,
      whh.astype(jnp.float32), bhh.reshape(3 * hid, 1).astype(jnp.float32))

    return out.reshape(N, hid, W)


# speedup vs baseline: 1.1513x; 1.1513x over previous
import functools

import jax
import jax.numpy as jnp
import numpy as np
from jax.experimental import pallas as pl
from jax.experimental.pallas import tpu as pltpu


def _leaky(v):
    return jnp.maximum(v, 0.01 * v)


def _sigmoid(v):
    return 0.5 * (jnp.tanh(0.5 * v) + 1.0)


def _encdec_body(x_ref, m1_ref, b1_ref, m2_ref,
                 wih_ref, whh_ref, bhh_ref, out_ref, *, W, B):
    hid = whh_ref.shape[1]

    xn = x_ref[...]
    xT = jnp.transpose(xn.astype(jnp.bfloat16))
    x2 = xT.reshape(x_ref.shape[1] // W, W * B)
    y1 = _leaky((jnp.dot(m1_ref[...], x2, preferred_element_type=jnp.float32)
                 + b1_ref[...]).astype(jnp.bfloat16))
    y2 = _leaky(jnp.dot(m2_ref[...], y1,
                        preferred_element_type=jnp.float32).astype(jnp.bfloat16))

    gi = jnp.dot(wih_ref[...], y2,
                 preferred_element_type=jnp.float32)

    whh = whh_ref[...]
    bhh = bhh_ref[...]
    h = jnp.zeros((hid, B), jnp.float32)
    for t in range(W):
        gi_t = gi[:, t * B:(t + 1) * B]
        gh = jnp.dot(whh, h, preferred_element_type=jnp.float32) + bhh
        g = gi_t + gh
        r = _sigmoid(g[0:hid, :])
        z = _sigmoid(g[hid:2 * hid, :])
        n = jnp.tanh(gi_t[2 * hid:3 * hid, :] + r * gh[2 * hid:3 * hid, :])
        h = n + z * (h - n)
        out_ref[:, t * B:(t + 1) * B] = h


def kernel(x, w1, b1, w2, b2, wih, whh, bih, bhh):
    N, C0, H, W = x.shape
    C1 = w1.shape[0]
    C2 = w2.shape[0]
    hid = whh.shape[1]
    He = H + 2

    B = 1
    for cand in (256, 128, 64, 32, 16, 8, 4, 2):
        if N % cand == 0:
            B = cand
            break
    NB = N // B

    xt = x.reshape(NB, B, C0 * H * W)

    jj = np.arange(He)
    hh = np.arange(H)
    interior = ((jj >= 1) & (jj <= H)).astype(np.float32)
    e1 = np.stack([(jj[:, None] + kh - 2 == hh[None, :]).astype(np.float32)
                   for kh in range(3)])
    e1 = e1 * interior[None, :, None]
    s1 = np.zeros((C1, He, C1 * He + 1), np.float32)
    for c in range(C1):
        for j in range(He):
            s1[c, j, c * He + j] = 1.0
    e1s = np.einsum('kjh,cjr->krch', e1, s1)
    m1 = jnp.einsum('krch,cak->rah',
                    jnp.asarray(e1s), w1.astype(jnp.float32))
    m1 = m1.reshape(C1 * He + 1, C0 * H).astype(jnp.bfloat16)
    b1e_sel = np.zeros((C1 * He + 1, C1), np.float32)
    for c in range(C1):
        b1e_sel[c * He:(c + 1) * He, c] = interior
    b1e_pad = np.zeros((C1 * He + 1, 1), np.float32)
    b1e_pad[C1 * He, 0] = 1.0
    b1e = (jnp.asarray(b1e_sel) @ b1.astype(jnp.float32)[:, None]
           + jnp.asarray(b1e_pad))

    e2 = np.stack([(hh[:, None] + kh == jj[None, :]).astype(np.float32)
                   for kh in range(3)])
    s2 = np.zeros((C2, H, C2 * H + 1), np.float32)
    for c in range(C2):
        for h in range(H):
            s2[c, h, c * H + h] = 1.0
    e2s = np.einsum('khj,chr->krcj', e2, s2)
    m2_band = jnp.einsum('krcj,cak->raj',
                         jnp.asarray(e2s), w2.astype(jnp.float32))
    m2_band = m2_band.reshape(C2 * H + 1, C1 * He)
    b2_sel = np.zeros((C2 * H + 1, C2), np.float32)
    for c in range(C2):
        b2_sel[c * H:(c + 1) * H, c] = 1.0
    b2_col = jnp.asarray(b2_sel) @ b2.astype(jnp.float32)[:, None]
    corner = np.zeros((C2 * H + 1, 1), np.float32)
    corner[C2 * H, 0] = 1.0
    m2 = jnp.concatenate([m2_band, b2_col + jnp.asarray(corner)],
                         axis=1).astype(jnp.bfloat16)

    wih_aug = jnp.concatenate(
        [wih.astype(jnp.float32),
         bih.reshape(3 * hid, 1).astype(jnp.float32)],
        axis=1).astype(jnp.bfloat16)

    out = pl.pallas_call(
        functools.partial(_encdec_body, W=W, B=B),
        out_shape=jax.ShapeDtypeStruct((NB, hid, W * B), jnp.float32),
        grid=(NB,),
        in_specs=[
            pl.BlockSpec((None, B, C0 * H * W), lambda i: (i, 0, 0)),
            pl.BlockSpec((C1 * He + 1, C0 * H), lambda i: (0, 0)),
            pl.BlockSpec((C1 * He + 1, 1), lambda i: (0, 0)),
            pl.BlockSpec((C2 * H + 1, C1 * He + 1), lambda i: (0, 0)),
            pl.BlockSpec((3 * hid, C2 * H + 1), lambda i: (0, 0)),
            pl.BlockSpec((3 * hid, hid), lambda i: (0, 0)),
            pl.BlockSpec((3 * hid, 1), lambda i: (0, 0)),
        ],
        out_specs=pl.BlockSpec((None, hid, W * B), lambda i: (i, 0, 0)),
        compiler_params=pltpu.CompilerParams(
            dimension_semantics=("arbitrary",),
            vmem_limit_bytes=60000 * 1024),
    )(xt, m1, b1e, m2, wih_aug,
      whh.astype(jnp.float32), bhh.reshape(3 * hid, 1).astype(jnp.float32))

    out = out.reshape(NB, hid, W, B)
    out = jnp.transpose(out, (0, 3, 1, 2)).reshape(N, hid, W)
    return out
